# Initial kernel scaffold; baseline (speedup 1.0000x reference)
import jax
import jax.numpy as jnp
from jax.experimental import pallas as pl


def kernel(x_A, x_B, edge_r0, edge_r1, Wl0, bl0, W00, b00, W10, b10, Wl1, bl1, W01, b01, W11, b11, Wm0, bm0, Wm1, bm1, Wout, bout):
    raise NotImplementedError("write your pallas kernel here")


# speedup vs baseline: 3.3681x; 3.3681x over previous
import functools

import jax
import jax.numpy as jnp
from jax import lax
from jax.experimental import pallas as pl
from jax.experimental.pallas import tpu as pltpu
from jax.experimental.pallas import tpu_sc as plsc

N = 10000
D = 128
E = 320000

NC = 2
NS = 16
NW = NC * NS
CHUNK = 128
C_PER_W = 80
E_PAD = NW * C_PER_W * CHUNK
N_PAD = 10240
ROWS_PER_TILE = N_PAD // NS
WCHUNKS = ROWS_PER_TILE // CHUNK


def _sc_segment_sum(x_a, srcs, dsts, zrows):
    mesh = plsc.VectorSubcoreMesh(
        core_axis_name="c", subcore_axis_name="s", num_cores=NC, num_subcores=NS
    )

    @functools.partial(
        pl.kernel,
        out_type=jax.ShapeDtypeStruct((NC, N_PAD, D), jnp.float32),
        mesh=mesh,
        scratch_types=[
            pltpu.VMEM((C_PER_W, CHUNK), jnp.int32),
            pltpu.VMEM((C_PER_W, CHUNK), jnp.int32),
            pltpu.VMEM((CHUNK, D), jnp.float32),
            pltpu.MemorySpace.VMEM_SHARED((N_PAD, D), jnp.float32),
            pltpu.SemaphoreType.DMA,
        ],
    )
    def sc_kernel(xa_hbm, src_hbm, dst_hbm, z_hbm, out_hbm,
                  idx_s, idx_d, rows, agg_sh, sem):
        c = lax.axis_index("c")
        s = lax.axis_index("s")
        wid = c * NS + s

        pltpu.sync_copy(z_hbm, rows)
        for k in range(WCHUNKS):
            pltpu.sync_copy(rows, agg_sh.at[pl.ds(s * ROWS_PER_TILE + k * CHUNK, CHUNK)])

        pltpu.sync_copy(src_hbm.at[wid], idx_s)
        pltpu.sync_copy(dst_hbm.at[wid], idx_d)
        plsc.subcore_barrier()

        def body(j, carry):
            pltpu.async_copy(xa_hbm.at[idx_s.at[j]], rows, sem).wait()
            pltpu.sync_copy(rows, agg_sh.at[idx_d.at[j]], add=True)
            return carry

        lax.fori_loop(0, C_PER_W, body, 0)
        plsc.subcore_barrier()

        for k in range(WCHUNKS):
            off = s * ROWS_PER_TILE + k * CHUNK
            pltpu.sync_copy(agg_sh.at[pl.ds(off, CHUNK)], rows)
            pltpu.sync_copy(rows, out_hbm.at[c, pl.ds(off, CHUNK)])

    return sc_kernel(x_a, srcs, dsts, zrows)


BLK = 1000


def _tc_body(p_ref, xa_ref, wl_ref, wc_ref, wf_ref, b1_ref, bf_ref, o_ref):
    agg = p_ref[0] + p_ref[1]
    xa = xa_ref[...]
    t = (jnp.dot(agg, wl_ref[...], preferred_element_type=jnp.float32)
         + jnp.dot(xa, wc_ref[...], preferred_element_type=jnp.float32)
         + b1_ref[...])
    o_ref[...] = (jnp.dot(jnp.maximum(t, 0.0), wf_ref[...],
                          preferred_element_type=jnp.float32)
                  + bf_ref[...])


def _tc_fused(partials, x_a, wl_t, wc_t, wf_t, b1, bf):
    return pl.pallas_call(
        _tc_body,
        grid=(N // BLK,),
        in_specs=[
            pl.BlockSpec((NC, BLK, D), lambda j: (0, j, 0)),
            pl.BlockSpec((BLK, D), lambda j: (j, 0)),
            pl.BlockSpec((D, D), lambda j: (0, 0)),
            pl.BlockSpec((D, D), lambda j: (0, 0)),
            pl.BlockSpec((D, D), lambda j: (0, 0)),
            pl.BlockSpec((1, D), lambda j: (0, 0)),
            pl.BlockSpec((1, D), lambda j: (0, 0)),
        ],
        out_specs=pl.BlockSpec((BLK, D), lambda j: (j, 0)),
        out_shape=jax.ShapeDtypeStruct((N, D), jnp.float32),
    )(partials, x_a, wl_t, wc_t, wf_t, b1, bf)


def kernel(x_A, x_B, edge_r0, edge_r1,
           Wl0, bl0, W00, b00, W10, b10,
           Wl1, bl1, W01, b01, W11, b11,
           Wm0, bm0, Wm1, bm1, Wout, bout):
    src = edge_r1[1]
    dst = edge_r1[0]
    pad = E_PAD - E
    srcs = jnp.concatenate([src, jnp.zeros((pad,), jnp.int32)])
    dsts = jnp.concatenate([dst, jnp.full((pad,), N, jnp.int32)])
    srcs = srcs.reshape(NW, C_PER_W, CHUNK)
    dsts = dsts.reshape(NW, C_PER_W, CHUNK)
    zrows = jnp.zeros((CHUNK, D), jnp.float32)

    partials = _sc_segment_sum(x_A, srcs, dsts, zrows)

    eye = jnp.eye(D, dtype=jnp.float32)
    wl_t = Wl1.T
    wc_t = (W01 + W11).T + eye
    b1 = (bl1 + b01 + b11).reshape(1, D)
    wf_t = (Wout @ Wm1).T
    bf = (bm1 @ Wout.T + bout).reshape(1, D)

    return _tc_fused(partials, x_A, wl_t, wc_t, wf_t, b1, bf)
